# Initial kernel scaffold; baseline (speedup 1.0000x reference)
import jax
import jax.numpy as jnp
from jax.experimental import pallas as pl


def kernel(x1_elem_weights, x1_elem_fea, x1_self_idx, x1_nbr_idx, x1_cry_idx, x2_elem_weights, x2_elem_fea, x2_self_idx, x2_nbr_idx, x2_cry_idx, emb_W, emb_b, mg_gW1, mg_gb1, mg_gW2, mg_gb2, mg_mW1, mg_mb1, mg_mW2, mg_mb2, mg_pow, cg_gW1, cg_gb1, cg_gW2, cg_gb2, cg_mW1, cg_mb1, cg_mW2, cg_mb2, cg_pow, head_W, head_b, rescale):
    raise NotImplementedError("write your pallas kernel here")


# speedup vs baseline: 5.9015x; 5.9015x over previous
import functools
import jax
import jax.numpy as jnp
from jax import lax
from jax.experimental import pallas as pl
from jax.experimental.pallas import tpu as pltpu
from jax.experimental.pallas import tpu_sc as plsc

N_NODES = 10000
N_EDGES = 160000
N_CRY = 625
ELEM_IN = 16
FEA = 64
HID = 256
HEADS = 3

BN = 256
NWIN = (N_NODES + BN - 1) // BN
CE = 512
CC = 1000
NCC = N_NODES // CC
CP = 640

_F32 = jnp.float32


def _leaky(x):
    return jnp.where(x >= 0, x, 0.01 * x)


def _embed_body(ef_ref, ew_ref, W_ref, b_ref, out_ref):
    h = jnp.dot(ef_ref[...], W_ref[...], preferred_element_type=_F32) + b_ref[...]
    col = lax.broadcasted_iota(jnp.int32, (N_NODES, FEA), 1)
    out_ref[...] = jnp.where(col == FEA - 1, ew_ref[...], h)


def _embed(ef, ew, W64, b64):
    return pl.pallas_call(
        _embed_body,
        out_shape=jax.ShapeDtypeStruct((N_NODES, FEA), _F32),
    )(ef, ew, W64, b64)


_SC_NW = 32
_SC_BPW = N_EDGES // _SC_NW
_SC_W = 2 * FEA
_SC_CH = 800
_SC_CHUNKS = [(o, min(_SC_CH, _SC_BPW - o)) for o in range(0, _SC_BPW, _SC_CH)]


def _sc_gather(table128, idx):
    mesh = plsc.VectorSubcoreMesh(core_axis_name="c", subcore_axis_name="s",
                                  num_cores=2)

    @functools.partial(
        pl.kernel, mesh=mesh,
        out_type=jax.ShapeDtypeStruct((N_EDGES, _SC_W), _F32),
        scratch_types=[
            pltpu.VMEM((_SC_CH,), jnp.int32),
            pltpu.VMEM((_SC_CH, _SC_W), _F32),
            pltpu.SemaphoreType.DMA,
        ],
    )
    def k(table_hbm, idx_hbm, out_hbm, idx_v, rows_v, sem):
        wid = lax.axis_index("s") * 2 + lax.axis_index("c")
        for off, sz in _SC_CHUNKS:
            base = wid * _SC_BPW + off
            idx_d = idx_v.at[pl.ds(0, sz)]
            rows_d = rows_v.at[pl.ds(0, sz)]
            pltpu.sync_copy(idx_hbm.at[pl.ds(base, sz)], idx_d)
            pltpu.async_copy(table_hbm.at[idx_d], rows_d, sem).wait()
            pltpu.sync_copy(rows_d, out_hbm.at[pl.ds(base, sz)])

    return k(table128, idx)


def _edge_body(sidx2d_ref, sidx_hbm, hn_hbm, hwin_ref,
               gW1_ref, gb1_ref, gW2_ref, gb2_ref,
               mW1_ref, mb1_ref, mW2_ref, mb2_ref, pow_ref,
               out_ref, sbuf, hnbuf, sem_s, sem_h):
    w = pl.program_id(0)
    ids2d = sidx2d_ref[...]
    base = jnp.sum((ids2d < w * BN).astype(jnp.int32))
    end = jnp.sum((ids2d < (w + 1) * BN).astype(jnp.int32))
    nchunks = (end - base + CE - 1) // CE

    z1 = jnp.zeros((BN, FEA), _F32)
    z0 = jnp.zeros((BN, 1), _F32)

    def body(t, carry):
        start = base + t * CE
        start_c = jnp.minimum(start, N_EDGES - CE)
        cp_s = pltpu.make_async_copy(sidx_hbm.at[pl.ds(start_c, CE), :], sbuf, sem_s)
        cp_h = pltpu.make_async_copy(hn_hbm.at[pl.ds(start_c, CE), :], hnbuf, sem_h)
        cp_s.start()
        cp_h.start()
        cp_s.wait()
        s_loc = sbuf[...] - w * BN
        pos = start_c + lax.broadcasted_iota(jnp.int32, (CE, 1), 0)
        valid = (pos >= start) & (pos < end)
        oh = (s_loc == lax.broadcasted_iota(jnp.int32, (CE, BN), 1)).astype(_F32)
        hs = jnp.dot(oh, hwin_ref[...], preferred_element_type=_F32)
        cp_h.wait()
        hn = hnbuf[...][:, :FEA]
        logw = jnp.log(hn[:, FEA - 1:FEA])
        new = []
        for i in range(HEADS):
            pg = (jnp.dot(hs, gW1_ref[i, :FEA, :], preferred_element_type=_F32)
                  + jnp.dot(hn, gW1_ref[i, FEA:, :], preferred_element_type=_F32)
                  + gb1_ref[i])
            g = jnp.dot(_leaky(pg), gW2_ref[i], preferred_element_type=_F32) + gb2_ref[i]
            pm = (jnp.dot(hs, mW1_ref[i, :FEA, :], preferred_element_type=_F32)
                  + jnp.dot(hn, mW1_ref[i, FEA:, :], preferred_element_type=_F32)
                  + mb1_ref[i])
            m = jnp.dot(_leaky(pm), mW2_ref[i], preferred_element_type=_F32) + mb2_ref[i]
            q = jnp.where(valid, jnp.exp(pow_ref[i] * logw + g), 0.0)
            a1, a0 = carry[2 * i], carry[2 * i + 1]
            dn = (((0,), (0,)), ((), ()))
            a1 = a1 + lax.dot_general(oh, q * m, dn, preferred_element_type=_F32)
            a0 = a0 + lax.dot_general(oh, q, dn, preferred_element_type=_F32)
            new.extend([a1, a0])
        return tuple(new)

    carry = lax.fori_loop(0, nchunks, body, (z1, z0) * HEADS)
    outv = hwin_ref[...]
    for i in range(HEADS):
        outv = outv + (1.0 / HEADS) * carry[2 * i] / (carry[2 * i + 1] + 1e-10)
    out_ref[...] = outv


def _edge_pass(h, sidx, sidx2d, hn, gW1, gb1, gW2, gb2, mW1, mb1, mW2, mb2, pw):
    full = lambda shp: pl.BlockSpec(shp, lambda w: (0,) * len(shp))
    return pl.pallas_call(
        _edge_body,
        grid=(NWIN,),
        in_specs=[
            full((N_EDGES // 128, 128)),
            pl.BlockSpec(memory_space=pl.ANY),
            pl.BlockSpec(memory_space=pl.ANY),
            pl.BlockSpec((BN, FEA), lambda w: (w, 0)),
            full((HEADS, 2 * FEA, HID)),
            full((HEADS, HID)),
            full((HEADS, HID, 1)),
            full((HEADS, 1)),
            full((HEADS, 2 * FEA, HID)),
            full((HEADS, HID)),
            full((HEADS, HID, FEA)),
            full((HEADS, FEA)),
            pl.BlockSpec(memory_space=pltpu.SMEM),
        ],
        out_specs=pl.BlockSpec((BN, FEA), lambda w: (w, 0)),
        out_shape=jax.ShapeDtypeStruct((NWIN * BN, FEA), _F32),
        scratch_shapes=[
            pltpu.VMEM((CE, 1), jnp.int32),
            pltpu.VMEM((CE, _SC_W), _F32),
            pltpu.SemaphoreType.DMA,
            pltpu.SemaphoreType.DMA,
        ],
    )(sidx2d, sidx, hn, h, gW1, gb1, gW2, gb2, mW1, mb1, mW2, mb2, pw)


def _cry_body(x_ref, ew_ref, cidx_ref,
              gW1_ref, gb1_ref, gW2_ref, gb2_ref,
              mW1_ref, mb1_ref, mW2_ref, mb2_ref, pow_ref,
              hW_ref, hb_ref, out_ref):
    acc1 = [jnp.zeros((CP, FEA), _F32) for _ in range(HEADS)]
    acc0 = [jnp.zeros((CP, 1), _F32) for _ in range(HEADS)]
    for c in range(NCC):
        x = x_ref[c * CC:(c + 1) * CC, :]
        logw = jnp.log(ew_ref[c * CC:(c + 1) * CC, :])
        ids = cidx_ref[c * CC:(c + 1) * CC, :]
        oh = (ids == lax.broadcasted_iota(jnp.int32, (CC, CP), 1)).astype(_F32)
        for i in range(HEADS):
            pg = jnp.dot(x, gW1_ref[i], preferred_element_type=_F32) + gb1_ref[i]
            g = jnp.dot(_leaky(pg), gW2_ref[i], preferred_element_type=_F32) + gb2_ref[i]
            pm = jnp.dot(x, mW1_ref[i], preferred_element_type=_F32) + mb1_ref[i]
            m = jnp.dot(_leaky(pm), mW2_ref[i], preferred_element_type=_F32) + mb2_ref[i]
            q = jnp.exp(pow_ref[i] * logw + g)
            dn = (((0,), (0,)), ((), ()))
            acc1[i] = acc1[i] + lax.dot_general(oh, q * m, dn, preferred_element_type=_F32)
            acc0[i] = acc0[i] + lax.dot_general(oh, q, dn, preferred_element_type=_F32)
    cry = jnp.zeros((CP, FEA), _F32)
    for i in range(HEADS):
        cry = cry + (1.0 / HEADS) * acc1[i] / (acc0[i] + 1e-10)
    out_ref[...] = jnp.dot(cry, hW_ref[...], preferred_element_type=_F32) + hb_ref[...]


def _cry_pool(x, ew, cidx_col, gW1, gb1, gW2, gb2, mW1, mb1, mW2, mb2, pw, hW, hb):
    in_specs = [pl.BlockSpec(memory_space=pltpu.SMEM) if k == 11 else pl.BlockSpec()
                for k in range(14)]
    return pl.pallas_call(
        _cry_body,
        in_specs=in_specs,
        out_shape=jax.ShapeDtypeStruct((CP, FEA), _F32),
    )(x, ew, cidx_col, gW1, gb1, gW2, gb2, mW1, mb1, mW2, mb2, pw, hW, hb)


def _dist_body(z1_ref, z2_ref, rs_ref, out_ref):
    d = z1_ref[...] - z2_ref[...]
    s = jnp.sqrt(jnp.sum(d * d, axis=1))
    out_ref[...] = jnp.tanh(jnp.clip(s * jnp.exp(rs_ref[0]), 0.0, 5.0))


def _dist(z1, z2, rescale):
    return pl.pallas_call(
        _dist_body,
        in_specs=[pl.BlockSpec(), pl.BlockSpec(),
                  pl.BlockSpec(memory_space=pltpu.SMEM)],
        out_shape=jax.ShapeDtypeStruct((CP,), _F32),
    )(z1, z2, rescale)


def kernel(x1_elem_weights, x1_elem_fea, x1_self_idx, x1_nbr_idx, x1_cry_idx,
           x2_elem_weights, x2_elem_fea, x2_self_idx, x2_nbr_idx, x2_cry_idx,
           emb_W, emb_b,
           mg_gW1, mg_gb1, mg_gW2, mg_gb2, mg_mW1, mg_mb1, mg_mW2, mg_mb2, mg_pow,
           cg_gW1, cg_gb1, cg_gW2, cg_gb2, cg_mW1, cg_mb1, cg_mW2, cg_mb2, cg_pow,
           head_W, head_b, rescale):
    W64 = jnp.pad(emb_W, ((0, 0), (0, 1)))
    b64 = jnp.pad(emb_b, (0, 1)).reshape(1, FEA)

    def graph(ew, ef, sidx, nidx, cidx):
        sidx = sidx.astype(jnp.int32)
        nidx = nidx.astype(jnp.int32)
        cidx = cidx.astype(jnp.int32)
        h = _embed(ef, ew, W64, b64)
        hn = _sc_gather(jnp.pad(h, ((0, 0), (0, _SC_W - FEA))), nidx)
        h_pad = jnp.pad(h, ((0, NWIN * BN - N_NODES), (0, 0)))
        m_out = _edge_pass(h_pad, sidx.reshape(N_EDGES, 1),
                           sidx.reshape(N_EDGES // 128, 128), hn,
                           mg_gW1, mg_gb1, mg_gW2, mg_gb2,
                           mg_mW1, mg_mb1, mg_mW2, mg_mb2, mg_pow)
        z = _cry_pool(m_out[:N_NODES], ew, cidx.reshape(N_NODES, 1),
                      cg_gW1, cg_gb1, cg_gW2, cg_gb2,
                      cg_mW1, cg_mb1, cg_mW2, cg_mb2, cg_pow, head_W, head_b)
        return z

    z1 = graph(x1_elem_weights, x1_elem_fea, x1_self_idx, x1_nbr_idx, x1_cry_idx)
    z2 = graph(x2_elem_weights, x2_elem_fea, x2_self_idx, x2_nbr_idx, x2_cry_idx)
    return _dist(z1, z2, rescale)[:N_CRY]


# speedup vs baseline: 8.1559x; 1.3820x over previous
import functools
import jax
import jax.numpy as jnp
from jax import lax
from jax.experimental import pallas as pl
from jax.experimental.pallas import tpu as pltpu
from jax.experimental.pallas import tpu_sc as plsc

N_NODES = 10000
N_EDGES = 160000
N_CRY = 625
ELEM_IN = 16
FEA = 64
HID = 256
HEADS = 3

BN = 256
NWIN = (N_NODES + BN - 1) // BN
CE = 1024
CC = 1000
NCC = N_NODES // CC
CP = 640

_F32 = jnp.float32


def _leaky(x):
    return jnp.where(x >= 0, x, 0.01 * x)


def _embed_body(ef_ref, ew_ref, W_ref, b_ref, out_ref):
    h = jnp.dot(ef_ref[...], W_ref[...], preferred_element_type=_F32) + b_ref[...]
    col = lax.broadcasted_iota(jnp.int32, (N_NODES, FEA), 1)
    out_ref[...] = jnp.where(col == FEA - 1, ew_ref[...], h)


def _embed(ef, ew, W64, b64):
    return pl.pallas_call(
        _embed_body,
        out_shape=jax.ShapeDtypeStruct((N_NODES, FEA), _F32),
    )(ef, ew, W64, b64)


_SC_NW = 32
_SC_BPW = N_EDGES // _SC_NW
_SC_W = 2 * FEA
_SC_CH = 800
_SC_CHUNKS = [(o, min(_SC_CH, _SC_BPW - o)) for o in range(0, _SC_BPW, _SC_CH)]


def _sc_gather(table128, idx):
    mesh = plsc.VectorSubcoreMesh(core_axis_name="c", subcore_axis_name="s",
                                  num_cores=2)

    @functools.partial(
        pl.kernel, mesh=mesh,
        out_type=jax.ShapeDtypeStruct((N_EDGES, _SC_W), _F32),
        scratch_types=[
            pltpu.VMEM((_SC_CH,), jnp.int32),
            pltpu.VMEM((_SC_CH, _SC_W), _F32),
            pltpu.SemaphoreType.DMA,
        ],
    )
    def k(table_hbm, idx_hbm, out_hbm, idx_v, rows_v, sem):
        wid = lax.axis_index("s") * 2 + lax.axis_index("c")
        for off, sz in _SC_CHUNKS:
            base = wid * _SC_BPW + off
            idx_d = idx_v.at[pl.ds(0, sz)]
            rows_d = rows_v.at[pl.ds(0, sz)]
            pltpu.sync_copy(idx_hbm.at[pl.ds(base, sz)], idx_d)
            pltpu.async_copy(table_hbm.at[idx_d], rows_d, sem).wait()
            pltpu.sync_copy(rows_d, out_hbm.at[pl.ds(base, sz)])

    return k(table128, idx)


def _edge_body(sidx2d_ref, sidx_hbm, hn_hbm, hwin_ref,
               gW1_ref, gb1_ref, gW2_ref, gb2_ref,
               mW1_ref, mb1_ref, mW2_ref, mb2_ref, pow_ref,
               out_ref, sbuf, hnbuf, sem_s, sem_h):
    w = pl.program_id(0)
    ids2d = sidx2d_ref[...]
    base = jnp.sum((ids2d < w * BN).astype(jnp.int32))
    end = jnp.sum((ids2d < (w + 1) * BN).astype(jnp.int32))
    nchunks = (end - base + CE - 1) // CE

    z1 = jnp.zeros((BN, FEA), _F32)
    z0 = jnp.zeros((BN, 1), _F32)

    def _copies(t):
        start = base + t * CE
        start_c = jnp.minimum(start, N_EDGES - CE)
        slot = lax.rem(t, 2)
        cp_s = pltpu.make_async_copy(sidx_hbm.at[pl.ds(start_c, CE), :],
                                     sbuf.at[slot], sem_s.at[slot])
        cp_h = pltpu.make_async_copy(hn_hbm.at[pl.ds(start_c, CE), :],
                                     hnbuf.at[slot], sem_h.at[slot])
        return cp_s, cp_h

    @pl.when(nchunks > 0)
    def _():
        cp_s, cp_h = _copies(0)
        cp_s.start()
        cp_h.start()

    def body(t, carry):
        start = base + t * CE
        start_c = jnp.minimum(start, N_EDGES - CE)
        slot = lax.rem(t, 2)

        @pl.when(t + 1 < nchunks)
        def _():
            cp_s, cp_h = _copies(t + 1)
            cp_s.start()
            cp_h.start()

        cp_s, cp_h = _copies(t)
        cp_s.wait()
        s_loc = sbuf[slot] - w * BN
        pos = start_c + lax.broadcasted_iota(jnp.int32, (CE, 1), 0)
        valid = (pos >= start) & (pos < end)
        oh = (s_loc == lax.broadcasted_iota(jnp.int32, (CE, BN), 1)).astype(_F32)
        hs = jnp.dot(oh, hwin_ref[...], preferred_element_type=_F32)
        cp_h.wait()
        hn = hnbuf[slot][:, :FEA]
        logw = jnp.log(hn[:, FEA - 1:FEA])
        new = []
        for i in range(HEADS):
            pg = (jnp.dot(hs, gW1_ref[i, :FEA, :], preferred_element_type=_F32)
                  + jnp.dot(hn, gW1_ref[i, FEA:, :], preferred_element_type=_F32)
                  + gb1_ref[i])
            g = jnp.dot(_leaky(pg), gW2_ref[i], preferred_element_type=_F32) + gb2_ref[i]
            pm = (jnp.dot(hs, mW1_ref[i, :FEA, :], preferred_element_type=_F32)
                  + jnp.dot(hn, mW1_ref[i, FEA:, :], preferred_element_type=_F32)
                  + mb1_ref[i])
            m = jnp.dot(_leaky(pm), mW2_ref[i], preferred_element_type=_F32) + mb2_ref[i]
            q = jnp.where(valid, jnp.exp(pow_ref[i] * logw + g), 0.0)
            a1, a0 = carry[2 * i], carry[2 * i + 1]
            dn = (((0,), (0,)), ((), ()))
            a1 = a1 + lax.dot_general(oh, q * m, dn, preferred_element_type=_F32)
            a0 = a0 + lax.dot_general(oh, q, dn, preferred_element_type=_F32)
            new.extend([a1, a0])
        return tuple(new)

    carry = lax.fori_loop(0, nchunks, body, (z1, z0) * HEADS)
    outv = hwin_ref[...]
    for i in range(HEADS):
        outv = outv + (1.0 / HEADS) * carry[2 * i] / (carry[2 * i + 1] + 1e-10)
    out_ref[...] = outv


def _edge_pass(h, sidx, sidx2d, hn, gW1, gb1, gW2, gb2, mW1, mb1, mW2, mb2, pw):
    full = lambda shp: pl.BlockSpec(shp, lambda w: (0,) * len(shp))
    return pl.pallas_call(
        _edge_body,
        grid=(NWIN,),
        in_specs=[
            full((N_EDGES // 128, 128)),
            pl.BlockSpec(memory_space=pl.ANY),
            pl.BlockSpec(memory_space=pl.ANY),
            pl.BlockSpec((BN, FEA), lambda w: (w, 0)),
            full((HEADS, 2 * FEA, HID)),
            full((HEADS, HID)),
            full((HEADS, HID, 1)),
            full((HEADS, 1)),
            full((HEADS, 2 * FEA, HID)),
            full((HEADS, HID)),
            full((HEADS, HID, FEA)),
            full((HEADS, FEA)),
            pl.BlockSpec(memory_space=pltpu.SMEM),
        ],
        out_specs=pl.BlockSpec((BN, FEA), lambda w: (w, 0)),
        out_shape=jax.ShapeDtypeStruct((NWIN * BN, FEA), _F32),
        scratch_shapes=[
            pltpu.VMEM((2, CE, 1), jnp.int32),
            pltpu.VMEM((2, CE, _SC_W), _F32),
            pltpu.SemaphoreType.DMA((2,)),
            pltpu.SemaphoreType.DMA((2,)),
        ],
    )(sidx2d, sidx, hn, h, gW1, gb1, gW2, gb2, mW1, mb1, mW2, mb2, pw)


def _cry_body(x_ref, ew_ref, cidx_ref,
              gW1_ref, gb1_ref, gW2_ref, gb2_ref,
              mW1_ref, mb1_ref, mW2_ref, mb2_ref, pow_ref,
              hW_ref, hb_ref, out_ref):
    acc1 = [jnp.zeros((CP, FEA), _F32) for _ in range(HEADS)]
    acc0 = [jnp.zeros((CP, 1), _F32) for _ in range(HEADS)]
    for c in range(NCC):
        x = x_ref[c * CC:(c + 1) * CC, :]
        logw = jnp.log(ew_ref[c * CC:(c + 1) * CC, :])
        ids = cidx_ref[c * CC:(c + 1) * CC, :]
        oh = (ids == lax.broadcasted_iota(jnp.int32, (CC, CP), 1)).astype(_F32)
        for i in range(HEADS):
            pg = jnp.dot(x, gW1_ref[i], preferred_element_type=_F32) + gb1_ref[i]
            g = jnp.dot(_leaky(pg), gW2_ref[i], preferred_element_type=_F32) + gb2_ref[i]
            pm = jnp.dot(x, mW1_ref[i], preferred_element_type=_F32) + mb1_ref[i]
            m = jnp.dot(_leaky(pm), mW2_ref[i], preferred_element_type=_F32) + mb2_ref[i]
            q = jnp.exp(pow_ref[i] * logw + g)
            dn = (((0,), (0,)), ((), ()))
            acc1[i] = acc1[i] + lax.dot_general(oh, q * m, dn, preferred_element_type=_F32)
            acc0[i] = acc0[i] + lax.dot_general(oh, q, dn, preferred_element_type=_F32)
    cry = jnp.zeros((CP, FEA), _F32)
    for i in range(HEADS):
        cry = cry + (1.0 / HEADS) * acc1[i] / (acc0[i] + 1e-10)
    out_ref[...] = jnp.dot(cry, hW_ref[...], preferred_element_type=_F32) + hb_ref[...]


def _cry_pool(x, ew, cidx_col, gW1, gb1, gW2, gb2, mW1, mb1, mW2, mb2, pw, hW, hb):
    in_specs = [pl.BlockSpec(memory_space=pltpu.SMEM) if k == 11 else pl.BlockSpec()
                for k in range(14)]
    return pl.pallas_call(
        _cry_body,
        in_specs=in_specs,
        out_shape=jax.ShapeDtypeStruct((CP, FEA), _F32),
    )(x, ew, cidx_col, gW1, gb1, gW2, gb2, mW1, mb1, mW2, mb2, pw, hW, hb)


def _dist_body(z1_ref, z2_ref, rs_ref, out_ref):
    d = z1_ref[...] - z2_ref[...]
    s = jnp.sqrt(jnp.sum(d * d, axis=1))
    out_ref[...] = jnp.tanh(jnp.clip(s * jnp.exp(rs_ref[0]), 0.0, 5.0))


def _dist(z1, z2, rescale):
    return pl.pallas_call(
        _dist_body,
        in_specs=[pl.BlockSpec(), pl.BlockSpec(),
                  pl.BlockSpec(memory_space=pltpu.SMEM)],
        out_shape=jax.ShapeDtypeStruct((CP,), _F32),
    )(z1, z2, rescale)


def kernel(x1_elem_weights, x1_elem_fea, x1_self_idx, x1_nbr_idx, x1_cry_idx,
           x2_elem_weights, x2_elem_fea, x2_self_idx, x2_nbr_idx, x2_cry_idx,
           emb_W, emb_b,
           mg_gW1, mg_gb1, mg_gW2, mg_gb2, mg_mW1, mg_mb1, mg_mW2, mg_mb2, mg_pow,
           cg_gW1, cg_gb1, cg_gW2, cg_gb2, cg_mW1, cg_mb1, cg_mW2, cg_mb2, cg_pow,
           head_W, head_b, rescale):
    W64 = jnp.pad(emb_W, ((0, 0), (0, 1)))
    b64 = jnp.pad(emb_b, (0, 1)).reshape(1, FEA)

    def graph(ew, ef, sidx, nidx, cidx):
        sidx = sidx.astype(jnp.int32)
        nidx = nidx.astype(jnp.int32)
        cidx = cidx.astype(jnp.int32)
        h = _embed(ef, ew, W64, b64)
        hn = _sc_gather(jnp.pad(h, ((0, 0), (0, _SC_W - FEA))), nidx)
        h_pad = jnp.pad(h, ((0, NWIN * BN - N_NODES), (0, 0)))
        m_out = _edge_pass(h_pad, sidx.reshape(N_EDGES, 1),
                           sidx.reshape(N_EDGES // 128, 128), hn,
                           mg_gW1, mg_gb1, mg_gW2, mg_gb2,
                           mg_mW1, mg_mb1, mg_mW2, mg_mb2, mg_pow)
        z = _cry_pool(m_out[:N_NODES], ew, cidx.reshape(N_NODES, 1),
                      cg_gW1, cg_gb1, cg_gW2, cg_gb2,
                      cg_mW1, cg_mb1, cg_mW2, cg_mb2, cg_pow, head_W, head_b)
        return z

    z1 = graph(x1_elem_weights, x1_elem_fea, x1_self_idx, x1_nbr_idx, x1_cry_idx)
    z2 = graph(x2_elem_weights, x2_elem_fea, x2_self_idx, x2_nbr_idx, x2_cry_idx)
    return _dist(z1, z2, rescale)[:N_CRY]
